# Initial kernel scaffold; baseline (speedup 1.0000x reference)
import jax
import jax.numpy as jnp
from jax.experimental import pallas as pl


def kernel(z, pos, batch, edge_index, emb, W_msg, proj_W, proj_b):
    raise NotImplementedError("write your pallas kernel here")


# speedup vs baseline: 9.1526x; 9.1526x over previous
import functools

import jax
import jax.numpy as jnp
from jax import lax
from jax.experimental import pallas as pl
from jax.experimental.pallas import tpu as pltpu
from jax.experimental.pallas import tpu_sc as plsc

G, M, DEG, H, OUT, CUTOFF = 250, 40, 16, 128, 12, 5.0
N = G * M
E = N * DEG
NC, NS = 2, 16
NW = NC * NS
NPW = 313
NPAD = NW * NPW
CPW = NPW * M
CPW16 = ((CPW + 15) // 16) * 16
GB = 10
GSTEPS = G // GB


def _sc_count_body(dst_hbm, out_hbm, dst_v, c_v):
    wid = lax.axis_index("s") * NC + lax.axis_index("c")
    pltpu.sync_copy(dst_hbm.at[pl.ds(wid * NPW * DEG, NPW * DEG)], dst_v)

    zeros16 = jnp.zeros((16,), jnp.float32)

    def zero_body(i, _):
        c_v[pl.ds(i * 16, 16)] = zeros16
        return 0

    lax.fori_loop(0, CPW16 // 16, zero_body, 0)

    ones16 = jnp.ones((16,), jnp.float32)

    def node_body(i, _):
        dvec = dst_v[pl.ds(i * DEG, DEG)]
        lj = lax.rem(dvec, M)
        plsc.addupdate_scatter(c_v, [lj + i * M], ones16)
        return 0

    lax.fori_loop(0, NPW, node_body, 0)
    pltpu.sync_copy(c_v.at[pl.ds(0, CPW)], out_hbm.at[pl.ds(wid * CPW, CPW)])


@functools.cache
def _sc_count():
    return pl.kernel(
        _sc_count_body,
        out_type=jax.ShapeDtypeStruct((NPAD * M,), jnp.float32),
        mesh=plsc.VectorSubcoreMesh(core_axis_name="c", subcore_axis_name="s",
                                    num_cores=NC, num_subcores=NS),
        compiler_params=pltpu.CompilerParams(needs_layout_passes=False),
        scratch_types=[
            pltpu.VMEM((NPW * DEG,), jnp.int32),
            pltpu.VMEM((CPW16,), jnp.float32),
        ],
    )

_DOT = dict(preferred_element_type=jnp.float32, precision=lax.Precision.HIGHEST)


def _tc_main_body(z_ref, pt_ref, cnt_ref, emb_ref, wmsg_ref, pw_ref, pb_ref,
                  out_ref):
    embv = emb_ref[...]
    wmsg = wmsg_ref[...]
    eye = (lax.broadcasted_iota(jnp.int32, (M, M), 0) ==
           lax.broadcasted_iota(jnp.int32, (M, M), 1)).astype(jnp.float32)
    kiota = lax.broadcasted_iota(jnp.int32, (100, M), 0)
    pb = pb_ref[...]

    for i in range(GB):
        zrow = z_ref[0, i:i + 1, :]
        onehot_t = (jnp.broadcast_to(zrow, (100, M)) == kiota
                    ).astype(jnp.float32)
        x0 = lax.dot_general(onehot_t, embv,
                             (((0,), (0,)), ((), ())), **_DOT)

        p_t = pt_ref[0, 3 * i:3 * i + 3, :]
        gram = lax.dot_general(p_t, p_t, (((0,), (0,)), ((), ())), **_DOT)
        sq_row = jnp.sum(p_t * p_t, axis=0, keepdims=True)
        sq_col = lax.dot_general(eye, sq_row,
                                 (((1,), (1,)), ((), ())), **_DOT)
        d2 = jnp.maximum(sq_col + sq_row - 2.0 * gram, 0.0)
        ew = jnp.sqrt(d2 + 1e-12)
        wm = 0.5 * (jnp.cos(jnp.pi * jnp.clip(ew / CUTOFF, 0.0, 1.0)) + 1.0)

        cg = cnt_ref[0, M * i:M * i + M, :]
        m1 = cg * wm
        agg = lax.dot_general(m1, x0, (((0,), (0,)), ((), ())), **_DOT)
        pre = x0 + lax.dot_general(agg, wmsg, (((1,), (0,)), ((), ())), **_DOT)
        x = pre * jax.nn.sigmoid(pre)

        cg_t = lax.dot_general(cg, eye, (((0,), (0,)), ((), ())), **_DOT)
        adj = ((cg + cg_t) > 0.0).astype(jnp.float32)

        walk = x
        o = pb + lax.dot_general(jnp.sum(walk, axis=0, keepdims=True),
                                 pw_ref[0], (((1,), (0,)), ((), ())), **_DOT)
        for k in range(5):
            walk = lax.dot_general(adj, walk,
                                   (((1,), (0,)), ((), ())), **_DOT) * x
            o = o + lax.dot_general(jnp.sum(walk, axis=0, keepdims=True),
                                    pw_ref[k + 1],
                                    (((1,), (0,)), ((), ())), **_DOT)
        out_ref[0, i:i + 1, :] = o


def kernel(z, pos, batch, edge_index, emb, W_msg, proj_W, proj_b):
    dst = edge_index[1]
    dst_pad = jnp.concatenate(
        [dst, jnp.full((NPAD * DEG - E,), N, jnp.int32)])
    cflat = _sc_count()(dst_pad)
    cnt3 = cflat.reshape(NPAD, M)[:N].reshape(GSTEPS, GB * M, M)

    z3 = z.reshape(GSTEPS, GB, M)
    pt3 = pos.reshape(G, M, 3).transpose(0, 2, 1).reshape(GSTEPS, GB * 3, M)
    pw3 = proj_W.reshape(6, H, OUT)
    pb2 = proj_b.reshape(1, OUT)

    out3 = pl.pallas_call(
        _tc_main_body,
        grid=(GSTEPS,),
        in_specs=[
            pl.BlockSpec((1, GB, M), lambda g: (g, 0, 0)),
            pl.BlockSpec((1, GB * 3, M), lambda g: (g, 0, 0)),
            pl.BlockSpec((1, GB * M, M), lambda g: (g, 0, 0)),
            pl.BlockSpec((100, H), lambda g: (0, 0)),
            pl.BlockSpec((H, H), lambda g: (0, 0)),
            pl.BlockSpec((6, H, OUT), lambda g: (0, 0, 0)),
            pl.BlockSpec((1, OUT), lambda g: (0, 0)),
        ],
        out_specs=pl.BlockSpec((1, GB, OUT), lambda g: (g, 0, 0)),
        out_shape=jax.ShapeDtypeStruct((GSTEPS, GB, OUT), jnp.float32),
    )(z3, pt3, cnt3, emb, W_msg, pw3, pb2)
    return out3.reshape(G, OUT)


# speedup vs baseline: 10.4598x; 1.1428x over previous
import functools

import jax
import jax.numpy as jnp
from jax import lax
from jax.experimental import pallas as pl
from jax.experimental.pallas import tpu as pltpu
from jax.experimental.pallas import tpu_sc as plsc

G, M, DEG, H, OUT, CUTOFF = 250, 40, 16, 128, 12, 5.0
N = G * M
E = N * DEG
NC, NS = 2, 16
NW = NC * NS
NPW = 313
NPAD = NW * NPW
CPW = NPW * M
CPW16 = ((CPW + 15) // 16) * 16
GB = 10
GSTEPS = G // GB


def _sc_count_body(dst_hbm, out_hbm, dst_v, c_v):
    wid = lax.axis_index("s") * NC + lax.axis_index("c")
    pltpu.sync_copy(dst_hbm.at[pl.ds(wid * NPW * DEG, NPW * DEG)], dst_v)

    zeros16 = jnp.zeros((16,), jnp.float32)

    def zero_body(i, _):
        c_v[pl.ds(i * 16, 16)] = zeros16
        return 0

    lax.fori_loop(0, CPW16 // 16, zero_body, 0)

    ones16 = jnp.ones((16,), jnp.float32)

    def node_body(i, _):
        dvec = dst_v[pl.ds(i * DEG, DEG)]
        lj = lax.rem(dvec, M)
        plsc.addupdate_scatter(c_v, [lj + i * M], ones16)
        return 0

    lax.fori_loop(0, NPW, node_body, 0)
    pltpu.sync_copy(c_v.at[pl.ds(0, CPW)], out_hbm.at[pl.ds(wid * CPW, CPW)])


@functools.cache
def _sc_count():
    return pl.kernel(
        _sc_count_body,
        out_type=jax.ShapeDtypeStruct((NPAD * M,), jnp.float32),
        mesh=plsc.VectorSubcoreMesh(core_axis_name="c", subcore_axis_name="s",
                                    num_cores=NC, num_subcores=NS),
        compiler_params=pltpu.CompilerParams(needs_layout_passes=False),
        scratch_types=[
            pltpu.VMEM((NPW * DEG,), jnp.int32),
            pltpu.VMEM((CPW16,), jnp.float32),
        ],
    )

_DOT = dict(preferred_element_type=jnp.float32, precision=lax.Precision.HIGHEST)


BM = GB * M


def _tc_main_body(z_ref, pt_ref, cnt_ref, emb_ref, wmsg_ref, pw_ref, pb_ref,
                  out_ref, c_scr):
    embv = emb_ref[...]
    wmsg = wmsg_ref[...]
    pb = pb_ref[...]

    @pl.when(pl.program_id(0) == 0)
    def _():
        c_scr[...] = jnp.zeros((BM, BM), jnp.float32)

    for i in range(GB):
        c_scr[M * i:M * i + M, M * i:M * i + M] = cnt_ref[0, M * i:M * i + M, :]

    cblk = c_scr[...]

    zall = z_ref[0]
    kiota = lax.broadcasted_iota(jnp.int32, (100, BM), 0)
    onehot_t = (jnp.broadcast_to(zall, (100, BM)) == kiota
                ).astype(jnp.float32)
    x0 = lax.dot_general(onehot_t, embv,
                         (((0,), (0,)), ((), ())), **_DOT)

    ptall = pt_ref[0]
    gram = lax.dot_general(ptall, ptall, (((0,), (0,)), ((), ())), **_DOT)
    sq_row = jnp.sum(ptall * ptall, axis=0, keepdims=True)
    eyebm = (lax.broadcasted_iota(jnp.int32, (BM, BM), 0) ==
             lax.broadcasted_iota(jnp.int32, (BM, BM), 1))
    eyef = eyebm.astype(jnp.float32)
    sq_col = lax.dot_general(eyef, sq_row,
                             (((1,), (1,)), ((), ())), **_DOT)
    d2 = jnp.maximum(sq_col + sq_row - 2.0 * gram, 0.0)
    ew = jnp.sqrt(d2 + 1e-12)
    wm = 0.5 * (jnp.cos(jnp.pi * jnp.clip(ew / CUTOFF, 0.0, 1.0)) + 1.0)

    m1 = cblk * wm
    agg = lax.dot_general(m1, x0, (((0,), (0,)), ((), ())), **_DOT)
    pre = x0 + lax.dot_general(agg, wmsg, (((1,), (0,)), ((), ())), **_DOT)
    x = pre * jax.nn.sigmoid(pre)

    cblk_t = lax.dot_general(
        cblk.astype(jnp.bfloat16), eyef.astype(jnp.bfloat16),
        (((0,), (0,)), ((), ())), preferred_element_type=jnp.float32)
    adj = ((cblk + cblk_t) > 0.0).astype(jnp.float32)

    sel = (lax.broadcasted_iota(jnp.int32, (GB, BM), 1) // M ==
           lax.broadcasted_iota(jnp.int32, (GB, BM), 0)).astype(jnp.float32)

    walk = x
    o = pb + lax.dot_general(
        lax.dot_general(sel, walk, (((1,), (0,)), ((), ())), **_DOT),
        pw_ref[0], (((1,), (0,)), ((), ())), **_DOT)
    for k in range(5):
        walk = lax.dot_general(adj, walk, (((1,), (0,)), ((), ())), **_DOT) * x
        o = o + lax.dot_general(
            lax.dot_general(sel, walk, (((1,), (0,)), ((), ())), **_DOT),
            pw_ref[k + 1], (((1,), (0,)), ((), ())), **_DOT)
    out_ref[0] = o


def kernel(z, pos, batch, edge_index, emb, W_msg, proj_W, proj_b):
    dst = edge_index[1]
    dst_pad = jnp.concatenate(
        [dst, jnp.full((NPAD * DEG - E,), N, jnp.int32)])
    cflat = _sc_count()(dst_pad)
    cnt3 = cflat.reshape(NPAD, M)[:N].reshape(GSTEPS, GB * M, M)

    z3 = z.reshape(GSTEPS, 1, BM)
    pt3 = pos.reshape(GSTEPS, GB, M, 3).transpose(0, 3, 1, 2).reshape(
        GSTEPS, 3, BM)
    pw3 = proj_W.reshape(6, H, OUT)
    pb2 = proj_b.reshape(1, OUT)

    out3 = pl.pallas_call(
        _tc_main_body,
        grid=(GSTEPS,),
        in_specs=[
            pl.BlockSpec((1, 1, BM), lambda g: (g, 0, 0)),
            pl.BlockSpec((1, 3, BM), lambda g: (g, 0, 0)),
            pl.BlockSpec((1, GB * M, M), lambda g: (g, 0, 0)),
            pl.BlockSpec((100, H), lambda g: (0, 0)),
            pl.BlockSpec((H, H), lambda g: (0, 0)),
            pl.BlockSpec((6, H, OUT), lambda g: (0, 0, 0)),
            pl.BlockSpec((1, OUT), lambda g: (0, 0)),
        ],
        out_specs=pl.BlockSpec((1, GB, OUT), lambda g: (g, 0, 0)),
        out_shape=jax.ShapeDtypeStruct((GSTEPS, GB, OUT), jnp.float32),
        scratch_shapes=[pltpu.VMEM((BM, BM), jnp.float32)],
    )(z3, pt3, cnt3, emb, W_msg, pw3, pb2)
    return out3.reshape(G, OUT)


# speedup vs baseline: 13.5871x; 1.2990x over previous
import functools

import jax
import jax.numpy as jnp
from jax import lax
from jax.experimental import pallas as pl
from jax.experimental.pallas import tpu as pltpu
from jax.experimental.pallas import tpu_sc as plsc

G, M, DEG, H, OUT, CUTOFF = 250, 40, 16, 128, 12, 5.0
N = G * M
E = N * DEG
NC, NS = 2, 16
NW = NC * NS
NPW = 313
NPAD = NW * NPW
CPW = NPW * M
CPW16 = ((CPW + 15) // 16) * 16
GB = 5
GSTEPS = G // GB


def _sc_count_body(dst_hbm, out_hbm, dst_v, c_v):
    wid = lax.axis_index("s") * NC + lax.axis_index("c")
    pltpu.sync_copy(dst_hbm.at[pl.ds(wid * NPW * DEG, NPW * DEG)], dst_v)

    zeros16 = jnp.zeros((16,), jnp.float32)

    def zero_body(i, _):
        c_v[pl.ds(i * 16, 16)] = zeros16
        return 0

    lax.fori_loop(0, CPW16 // 16, zero_body, 0)

    ones16 = jnp.ones((16,), jnp.float32)

    def node_body(i, _):
        dvec = dst_v[pl.ds(i * DEG, DEG)]
        lj = lax.rem(dvec, M)
        plsc.addupdate_scatter(c_v, [lj + i * M], ones16)
        return 0

    lax.fori_loop(0, NPW, node_body, 0)
    pltpu.sync_copy(c_v.at[pl.ds(0, CPW)], out_hbm.at[pl.ds(wid * CPW, CPW)])


@functools.cache
def _sc_count():
    return pl.kernel(
        _sc_count_body,
        out_type=jax.ShapeDtypeStruct((NPAD * M,), jnp.float32),
        mesh=plsc.VectorSubcoreMesh(core_axis_name="c", subcore_axis_name="s",
                                    num_cores=NC, num_subcores=NS),
        compiler_params=pltpu.CompilerParams(needs_layout_passes=False),
        scratch_types=[
            pltpu.VMEM((NPW * DEG,), jnp.int32),
            pltpu.VMEM((CPW16,), jnp.float32),
        ],
    )

_DOT = dict(preferred_element_type=jnp.float32, precision=lax.Precision.HIGHEST)


BM = GB * M


def _tc_main_body(z_ref, pt_ref, cnt_ref, emb_ref, wmsg_ref, pw_ref, pb_ref,
                  eye_ref, sel_ref, out_ref, c_scr):
    embv = emb_ref[...]
    wmsg = wmsg_ref[...]
    pb = pb_ref[...]
    eyef = eye_ref[...]
    sel = sel_ref[...]

    @pl.when(pl.program_id(0) == 0)
    def _():
        c_scr[...] = jnp.zeros((BM, BM), jnp.float32)

    for i in range(GB):
        c_scr[M * i:M * i + M, M * i:M * i + M] = cnt_ref[0, M * i:M * i + M, :]

    cblk = c_scr[...]

    zall = z_ref[0]
    kiota = lax.broadcasted_iota(jnp.int32, (100, BM), 0)
    onehot_t = (jnp.broadcast_to(zall, (100, BM)) == kiota
                ).astype(jnp.float32)
    x0 = lax.dot_general(onehot_t, embv,
                         (((0,), (0,)), ((), ())), **_DOT)

    ptall = pt_ref[0]
    gram = lax.dot_general(ptall, ptall, (((0,), (0,)), ((), ())), **_DOT)
    sq_row = jnp.sum(ptall * ptall, axis=0, keepdims=True)
    sq_col = lax.dot_general(eyef, sq_row,
                             (((1,), (1,)), ((), ())), **_DOT)
    d2 = jnp.maximum(sq_col + sq_row - 2.0 * gram, 0.0)
    ew = jnp.sqrt(d2 + 1e-12)
    wm = 0.5 * (jnp.cos(jnp.pi * jnp.clip(ew / CUTOFF, 0.0, 1.0)) + 1.0)

    m1 = cblk * wm
    agg = lax.dot_general(m1, x0, (((0,), (0,)), ((), ())), **_DOT)
    pre = x0 + lax.dot_general(agg, wmsg, (((1,), (0,)), ((), ())), **_DOT)
    x = pre * jax.nn.sigmoid(pre)

    cblk_t = lax.dot_general(
        cblk.astype(jnp.bfloat16), eyef.astype(jnp.bfloat16),
        (((0,), (0,)), ((), ())), preferred_element_type=jnp.float32)
    adj = ((cblk + cblk_t) > 0.0).astype(jnp.float32)

    walk = x
    o = pb + lax.dot_general(
        lax.dot_general(sel, walk, (((1,), (0,)), ((), ())), **_DOT),
        pw_ref[0], (((1,), (0,)), ((), ())), **_DOT)
    for k in range(5):
        walk = lax.dot_general(adj, walk, (((1,), (0,)), ((), ())), **_DOT) * x
        o = o + lax.dot_general(
            lax.dot_general(sel, walk, (((1,), (0,)), ((), ())), **_DOT),
            pw_ref[k + 1], (((1,), (0,)), ((), ())), **_DOT)
    out_ref[0] = o


def kernel(z, pos, batch, edge_index, emb, W_msg, proj_W, proj_b):
    dst = edge_index[1]
    dst_pad = jnp.concatenate(
        [dst, jnp.full((NPAD * DEG - E,), N, jnp.int32)])
    cflat = _sc_count()(dst_pad)
    cnt3 = cflat.reshape(NPAD, M)[:N].reshape(GSTEPS, GB * M, M)

    z3 = z.reshape(GSTEPS, 1, BM)
    pt3 = pos.reshape(GSTEPS, GB, M, 3).transpose(0, 3, 1, 2).reshape(
        GSTEPS, 3, BM)
    pw3 = proj_W.reshape(6, H, OUT)
    pb2 = proj_b.reshape(1, OUT)
    eyef = jnp.eye(BM, dtype=jnp.float32)
    sel = (jnp.arange(BM, dtype=jnp.int32)[None, :] // M ==
           jnp.arange(GB, dtype=jnp.int32)[:, None]).astype(jnp.float32)

    out3 = pl.pallas_call(
        _tc_main_body,
        grid=(GSTEPS,),
        in_specs=[
            pl.BlockSpec((1, 1, BM), lambda g: (g, 0, 0)),
            pl.BlockSpec((1, 3, BM), lambda g: (g, 0, 0)),
            pl.BlockSpec((1, GB * M, M), lambda g: (g, 0, 0)),
            pl.BlockSpec((100, H), lambda g: (0, 0)),
            pl.BlockSpec((H, H), lambda g: (0, 0)),
            pl.BlockSpec((6, H, OUT), lambda g: (0, 0, 0)),
            pl.BlockSpec((1, OUT), lambda g: (0, 0)),
            pl.BlockSpec((BM, BM), lambda g: (0, 0)),
            pl.BlockSpec((GB, BM), lambda g: (0, 0)),
        ],
        out_specs=pl.BlockSpec((1, GB, OUT), lambda g: (g, 0, 0)),
        out_shape=jax.ShapeDtypeStruct((GSTEPS, GB, OUT), jnp.float32),
        scratch_shapes=[pltpu.VMEM((BM, BM), jnp.float32)],
    )(z3, pt3, cnt3, emb, W_msg, pw3, pb2, eyef, sel)
    return out3.reshape(G, OUT)
